# Initial kernel scaffold; baseline (speedup 1.0000x reference)
import jax
import jax.numpy as jnp
from jax.experimental import pallas as pl


def kernel(x, edge_index_l0, edge_index_l1, W_l0, b_l0, W_r0, b_r0, W_l1, b_l1, W_r1, b_r1):
    raise NotImplementedError("write your pallas kernel here")


# speedup vs baseline: 2.9573x; 2.9573x over previous
import functools

import jax
import jax.numpy as jnp
from jax import lax
from jax.experimental import pallas as pl
from jax.experimental.pallas import tpu as pltpu
from jax.experimental.pallas import tpu_sc as plsc

N = 10000
D = 128
E = 320000

NC = 2
NS = 16
NW = NC * NS
EPW = E // NW
CH = 80
NCHUNK = EPW // CH
ZR = 80
NZCH = N // ZR


def _sc_agg_body(x_hbm, ei_hbm, acc_out,
                 src_v, dst_v, rows_v, acc_sh, sem):
    c = lax.axis_index("c")
    s = lax.axis_index("s")
    w = s * NC + c

    zero16 = jnp.zeros((16,), jnp.float32)
    one16 = jnp.ones((16,), jnp.float32)

    def _zrow(i, _):
        for j in range(D // 16):
            rows_v[i, pl.ds(j * 16, 16)] = zero16
        return 0
    lax.fori_loop(0, CH, _zrow, 0)

    lo = s * NZCH // NS
    hi = (s + 1) * NZCH // NS

    def _zero(i, _):
        pltpu.sync_copy(rows_v, acc_sh.at[pl.ds(i * ZR, ZR)])
        return 0
    lax.fori_loop(lo, hi, _zero, 0)
    plsc.subcore_barrier()

    def _edge(i, _):
        base = w * EPW + i * CH
        pltpu.sync_copy(ei_hbm.at[pl.ds(base, CH)], src_v)
        pltpu.sync_copy(ei_hbm.at[pl.ds(E + base, CH)], dst_v)
        pltpu.async_copy(x_hbm.at[src_v], rows_v, sem).wait()
        pltpu.sync_copy(rows_v, acc_sh.at[dst_v], add=True)
        return 0
    lax.fori_loop(0, NCHUNK, _edge, 0)
    plsc.subcore_barrier()


    def _out(i, _):
        r0 = i * ZR
        pltpu.sync_copy(acc_sh.at[pl.ds(r0, ZR)], rows_v)
        pltpu.sync_copy(rows_v, acc_out.at[c, pl.ds(r0, ZR)])
        return 0
    lax.fori_loop(lo, hi, _out, 0)


@functools.lru_cache(maxsize=1)
def _sc_agg():
    return pl.kernel(
        _sc_agg_body,
        out_type=[jax.ShapeDtypeStruct((NC, N, D), jnp.float32)],
        mesh=plsc.VectorSubcoreMesh(core_axis_name="c", subcore_axis_name="s"),
        scratch_types=[
            pltpu.VMEM((CH,), jnp.int32),
            pltpu.VMEM((CH,), jnp.int32),
            pltpu.VMEM((CH, D), jnp.float32),
            pltpu.VMEM_SHARED((N, D), jnp.float32),
            pltpu.SemaphoreType.DMA,
        ],
    )


def _dense_body(acc_ref, cnt_ref, x_ref, wl_ref, wr_ref, b_ref, o_ref, *, act):
    summed = acc_ref[0] + acc_ref[1]
    cnt = cnt_ref[0, :, 0:1] + cnt_ref[1, :, 0:1]
    mean = summed / jnp.maximum(cnt, 1.0)
    out = (jnp.dot(mean, wl_ref[...], preferred_element_type=jnp.float32)
           + jnp.dot(x_ref[...], wr_ref[...], preferred_element_type=jnp.float32)
           + b_ref[...])
    if act == "relu":
        o_ref[...] = jnp.maximum(out, 0.0)
    else:
        m = jnp.max(out, axis=-1, keepdims=True)
        lse = jnp.log(jnp.sum(jnp.exp(out - m), axis=-1, keepdims=True))
        o_ref[...] = out - m - lse


_RB = 2000


def _dense(acc, cnt, x, wl, wr, b, act):
    body = functools.partial(_dense_body, act=act)
    return pl.pallas_call(
        body,
        grid=(N // _RB,),
        in_specs=[
            pl.BlockSpec((NC, _RB, D), lambda i: (0, i, 0)),
            pl.BlockSpec((NC, _RB, D), lambda i: (0, i, 0)),
            pl.BlockSpec((_RB, D), lambda i: (i, 0)),
            pl.BlockSpec((D, D), lambda i: (0, 0)),
            pl.BlockSpec((D, D), lambda i: (0, 0)),
            pl.BlockSpec((1, D), lambda i: (0, 0)),
        ],
        out_specs=pl.BlockSpec((_RB, D), lambda i: (i, 0)),
        out_shape=jax.ShapeDtypeStruct((N, D), jnp.float32),
    )(acc, cnt, x, wl, wr, b)


def kernel(x, edge_index_l0, edge_index_l1, W_l0, b_l0, W_r0, b_r0,
           W_l1, b_l1, W_r1, b_r1):
    b0 = (b_l0 + b_r0).reshape(1, D)
    b1 = (b_l1 + b_r1).reshape(1, D)

    agg = _sc_agg()
    ones_nd = jnp.ones((N, D), jnp.float32)
    ei0 = edge_index_l0.reshape(-1)
    ei1 = edge_index_l1.reshape(-1)
    (acc0,) = agg(x, ei0)
    (cnt0,) = agg(ones_nd, ei0)
    h = _dense(acc0, cnt0, x, W_l0, W_r0, b0, "relu")
    (acc1,) = agg(h, ei1)
    (cnt1,) = agg(ones_nd, ei1)
    return _dense(acc1, cnt1, h, W_l1, W_r1, b1, "logsm")
